# baseline (speedup 1.0000x reference)
import functools

import jax
import jax.numpy as jnp
from jax import lax
from jax.experimental import pallas as pl
from jax.experimental.pallas import tpu as pltpu
from jax.experimental.pallas import tpu_sc as plsc

BATCH = 16384
DIM = 16
NC = 2
NS = 16
NW = NC * NS
BPW = BATCH // NW
CHUNK = 128
NCHUNK = BPW // CHUNK
NBLK = BPW // 16

_mesh = plsc.VectorSubcoreMesh(core_axis_name="c", subcore_axis_name="s")


@functools.partial(
    pl.kernel,
    mesh=_mesh,
    out_type=jax.ShapeDtypeStruct((NW, 16), jnp.float32),
    compiler_params=pltpu.CompilerParams(
        needs_layout_passes=False,
        use_tc_tiling_on_sc=False,
    ),
    scratch_types=[
        pltpu.VMEM((NCHUNK, CHUNK), jnp.int32),
        pltpu.VMEM((NCHUNK, CHUNK), jnp.int32),
        pltpu.VMEM((NCHUNK, CHUNK), jnp.int32),
        pltpu.VMEM((BPW, DIM), jnp.float32),
        pltpu.VMEM((BPW, DIM), jnp.float32),
        pltpu.VMEM((BPW, DIM), jnp.float32),
        pltpu.VMEM((16,), jnp.float32),
        pltpu.SemaphoreType.DMA,
        pltpu.SemaphoreType.DMA,
        pltpu.SemaphoreType.DMA,
    ],
)
def _sc_score(users_hbm, pos_hbm, neg_hbm, ut_hbm, it_hbm, out_hbm,
              iu, ip, ineg, ru, rp, rn, accv, su, sp_sem, sn_sem):
    wid = lax.axis_index("s") * NC + lax.axis_index("c")

    pltpu.sync_copy(users_hbm.at[wid], iu)
    pltpu.sync_copy(pos_hbm.at[wid], ip)
    pltpu.sync_copy(neg_hbm.at[wid], ineg)

    copies = []
    for j in range(NCHUNK):
        dst = pl.ds(j * CHUNK, CHUNK)
        copies.append(pltpu.async_copy(ut_hbm.at[iu.at[j]], ru.at[dst], su))
        copies.append(pltpu.async_copy(it_hbm.at[ip.at[j]], rp.at[dst], sp_sem))
        copies.append(pltpu.async_copy(it_hbm.at[ineg.at[j]], rn.at[dst], sn_sem))
    for c in copies:
        c.wait()

    lanes = lax.iota(jnp.int32, 16)
    zero = jnp.zeros((16,), jnp.float32)

    def block(k, acc):
        row = k * 16 + lanes
        sp = zero
        sn = zero
        for d in range(DIM):
            dv = jnp.full((16,), d, jnp.int32)
            uc = plsc.load_gather(ru, [row, dv])
            pc = plsc.load_gather(rp, [row, dv])
            nc = plsc.load_gather(rn, [row, dv])
            sp = sp + uc * pc
            sn = sn + uc * nc
        acc = acc + 1.0 / (1.0 + jnp.exp(-sp)) + 1.0 / (1.0 + jnp.exp(sn))
        return acc

    acc = lax.fori_loop(0, NBLK, block, zero)
    accv[...] = acc
    pltpu.sync_copy(accv, out_hbm.at[wid])


def _tc_reduce_body(p_ref, o_ref):
    o_ref[...] = (-jnp.sum(p_ref[...]) / BATCH).reshape(1, 1)


_tc_reduce = pl.pallas_call(
    _tc_reduce_body,
    out_shape=jax.ShapeDtypeStruct((1, 1), jnp.float32),
)


def kernel(users, pos_items, neg_items, user_table, item_table):
    u3 = users.astype(jnp.int32).reshape(NW, NCHUNK, CHUNK)
    p3 = pos_items.astype(jnp.int32).reshape(NW, NCHUNK, CHUNK)
    n3 = neg_items.reshape(-1).astype(jnp.int32).reshape(NW, NCHUNK, CHUNK)
    partials = _sc_score(u3, p3, n3, user_table, item_table)
    loss = _tc_reduce(partials)[0, 0]
    return (loss, loss, jnp.float32(0.0))
